# baseline (speedup 1.0000x reference)
import functools

import jax
import jax.numpy as jnp
from jax import lax
from jax.experimental import pallas as pl
from jax.experimental.pallas import tpu as pltpu
from jax.experimental.pallas import tpu_sc as plsc

NC = 2
NS = 16
NW = NC * NS
L = 16


@functools.lru_cache(maxsize=None)
def _build(B, F, V, D):
    assert D == NW, "one embedding dim per vector subcore"
    assert B % L == 0
    SPLIT = (V // 2) // 128 * 128
    VHI = V - SPLIT

    mesh = plsc.VectorSubcoreMesh(core_axis_name="c", subcore_axis_name="s")

    @functools.partial(
        pl.kernel,
        out_type=jax.ShapeDtypeStruct((F * D, B), jnp.float32),
        mesh=mesh,
        compiler_params=pltpu.CompilerParams(needs_layout_passes=False),
        scratch_types=[
            pltpu.VMEM((SPLIT,), jnp.float32),
            pltpu.VMEM((VHI,), jnp.float32),
            pltpu.VMEM((B,), jnp.float32),
            pltpu.SemaphoreType.DMA,
            pltpu.SemaphoreType.DMA,
            pltpu.SemaphoreType.DMA,
            pltpu.SemaphoreType.DMA,
        ],
    )
    def col_gather(cat_hbm, tab_hbm, out_hbm, lo_v, hi_v, buf_v,
                   sem_lo, sem_hi, sem_i, sem_o):
        cid = lax.axis_index("c")
        sid = lax.axis_index("s")
        d = sid * NC + cid

        def lo_src(f):
            return tab_hbm.at[f, d, pl.ds(0, SPLIT)]

        def hi_src(f):
            return tab_hbm.at[f, d, pl.ds(SPLIT, VHI)]

        def pass_lo(k, carry):
            ivf = buf_v[pl.ds(k * L, L)]
            iv = plsc.bitcast(ivf, jnp.int32)
            m = iv < SPLIT
            g = plsc.load_gather(lo_v, [iv], mask=m)
            buf_v[pl.ds(k * L, L)] = lax.select(m, g, ivf)
            return carry

        def pass_hi(k, carry):
            ivf = buf_v[pl.ds(k * L, L)]
            iv = plsc.bitcast(ivf, jnp.int32)
            m = (iv >= SPLIT) & (iv < V)
            g = plsc.load_gather(hi_v, [iv - SPLIT], mask=m)
            buf_v[pl.ds(k * L, L)] = lax.select(m, g, ivf)
            return carry

        pltpu.async_copy(lo_src(0), lo_v, sem_lo)
        pltpu.async_copy(hi_src(0), hi_v, sem_hi)
        pltpu.async_copy(cat_hbm.at[0, :], buf_v, sem_i)

        def do_field(f, carry):
            pltpu.make_async_copy(cat_hbm.at[f, :], buf_v, sem_i).wait()
            pltpu.make_async_copy(lo_src(f), lo_v, sem_lo).wait()
            lax.fori_loop(0, B // L, pass_lo, 0, unroll=16)

            @pl.when(f < F - 1)
            def _():
                pltpu.async_copy(lo_src(f + 1), lo_v, sem_lo)

            pltpu.make_async_copy(hi_src(f), hi_v, sem_hi).wait()
            lax.fori_loop(0, B // L, pass_hi, 0, unroll=16)

            @pl.when(f < F - 1)
            def _():
                pltpu.async_copy(hi_src(f + 1), hi_v, sem_hi)

            pltpu.async_copy(buf_v, out_hbm.at[f * D + d, :], sem_o)
            pltpu.make_async_copy(buf_v, out_hbm.at[f * D + d, :], sem_o).wait()

            @pl.when(f < F - 1)
            def _():
                pltpu.async_copy(cat_hbm.at[f + 1, :], buf_v, sem_i)
            return carry

        lax.fori_loop(0, F, do_field, 0)

    return col_gather


def kernel(categorical_features, tables):
    B, F = categorical_features.shape
    Ft, V, D = tables.shape
    assert Ft == F
    cat_f32 = lax.bitcast_convert_type(
        categorical_features.astype(jnp.int32), jnp.float32)
    cat_t = cat_f32.T
    tab_t = jnp.transpose(tables, (0, 2, 1))
    out_t = _build(B, F, V, D)(cat_t, tab_t)
    return out_t.T


# speedup vs baseline: 1.1934x; 1.1934x over previous
import functools

import jax
import jax.numpy as jnp
from jax import lax
from jax.experimental import pallas as pl
from jax.experimental.pallas import tpu as pltpu
from jax.experimental.pallas import tpu_sc as plsc

NC = 2
NS = 16
NW = NC * NS
L = 16


@functools.lru_cache(maxsize=None)
def _build(B, F, V, D):
    assert D == NW, "one embedding dim per vector subcore"
    BH = B // 2
    assert BH % L == 0

    mesh = plsc.VectorSubcoreMesh(core_axis_name="c", subcore_axis_name="s")

    @functools.partial(
        pl.kernel,
        out_type=jax.ShapeDtypeStruct((F * D, B), jnp.float32),
        mesh=mesh,
        compiler_params=pltpu.CompilerParams(needs_layout_passes=False),
        scratch_types=[
            pltpu.VMEM((V,), jnp.float32),
            pltpu.VMEM((BH,), jnp.float32),
            pltpu.VMEM((BH,), jnp.float32),
            pltpu.SemaphoreType.DMA,
            pltpu.SemaphoreType.DMA,
            pltpu.SemaphoreType.DMA,
            pltpu.SemaphoreType.DMA,
            pltpu.SemaphoreType.DMA,
        ],
    )
    def col_gather(cat_hbm, tab_hbm, out_hbm, slab_v, buf_a, buf_b,
                   sem_s, sem_ia, sem_ib, sem_oa, sem_ob):
        cid = lax.axis_index("c")
        sid = lax.axis_index("s")
        d = cid * NS + sid

        def idx_src(f, h):
            return cat_hbm.at[f, pl.ds(h * BH, BH)]

        def out_dst(f, h):
            return out_hbm.at[f * D + d, pl.ds(h * BH, BH)]

        def gather_half(buf):
            def gather16(k, carry):
                iv = plsc.bitcast(buf[pl.ds(k * L, L)], jnp.int32)
                buf[pl.ds(k * L, L)] = plsc.load_gather(slab_v, [iv])
                return carry

            lax.fori_loop(0, BH // L, gather16, 0, unroll=16)

        pltpu.async_copy(tab_hbm.at[0, d, :], slab_v, sem_s)
        pltpu.async_copy(idx_src(0, 0), buf_a, sem_ia)

        def do_field(f, carry):
            @pl.when(f > 0)
            def _():
                pltpu.make_async_copy(buf_b, out_dst(f - 1, 1), sem_ob).wait()
            pltpu.async_copy(idx_src(f, 1), buf_b, sem_ib)

            pltpu.make_async_copy(idx_src(f, 0), buf_a, sem_ia).wait()
            pltpu.make_async_copy(tab_hbm.at[f, d, :], slab_v, sem_s).wait()
            gather_half(buf_a)
            pltpu.async_copy(buf_a, out_dst(f, 0), sem_oa)

            pltpu.make_async_copy(idx_src(f, 1), buf_b, sem_ib).wait()
            gather_half(buf_b)

            @pl.when(f < F - 1)
            def _():
                pltpu.async_copy(tab_hbm.at[f + 1, d, :], slab_v, sem_s)
            pltpu.async_copy(buf_b, out_dst(f, 1), sem_ob)

            pltpu.make_async_copy(buf_a, out_dst(f, 0), sem_oa).wait()

            @pl.when(f < F - 1)
            def _():
                pltpu.async_copy(idx_src(f + 1, 0), buf_a, sem_ia)
            return carry

        lax.fori_loop(0, F, do_field, 0)
        pltpu.make_async_copy(buf_b, out_dst(F - 1, 1), sem_ob).wait()

    return col_gather


def kernel(categorical_features, tables):
    B, F = categorical_features.shape
    Ft, V, D = tables.shape
    assert Ft == F
    cat_f32 = lax.bitcast_convert_type(
        categorical_features.astype(jnp.int32), jnp.float32)
    cat_t = cat_f32.T
    tab_t = jnp.transpose(tables, (0, 2, 1))
    out_t = _build(B, F, V, D)(cat_t, tab_t)
    return out_t.T


# speedup vs baseline: 1.1952x; 1.0015x over previous
import functools

import jax
import jax.numpy as jnp
from jax import lax
from jax.experimental import pallas as pl
from jax.experimental.pallas import tpu as pltpu
from jax.experimental.pallas import tpu_sc as plsc

NC = 2
NS = 16
NW = NC * NS
L = 16


@functools.lru_cache(maxsize=None)
def _build(B, F, V, D):
    assert D == NW, "one embedding dim per vector subcore"
    BH = B // 2
    assert BH % L == 0

    mesh = plsc.VectorSubcoreMesh(core_axis_name="c", subcore_axis_name="s")

    @functools.partial(
        pl.kernel,
        out_type=jax.ShapeDtypeStruct((F * D, B), jnp.float32),
        mesh=mesh,
        compiler_params=pltpu.CompilerParams(needs_layout_passes=False),
        scratch_types=[
            pltpu.VMEM((V,), jnp.float32),
            pltpu.VMEM((BH,), jnp.float32),
            pltpu.VMEM((BH,), jnp.float32),
            pltpu.SemaphoreType.DMA,
            pltpu.SemaphoreType.DMA,
            pltpu.SemaphoreType.DMA,
            pltpu.SemaphoreType.DMA,
            pltpu.SemaphoreType.DMA,
        ],
    )
    def col_gather(cat_hbm, tab_hbm, out_hbm, slab_v, buf_a, buf_b,
                   sem_s, sem_ia, sem_ib, sem_oa, sem_ob):
        cid = lax.axis_index("c")
        sid = lax.axis_index("s")
        d = cid * NS + sid

        def issue_slab(f):
            pltpu.async_copy(tab_hbm.at[f, d, :], slab_v, sem_s)

        def wait_slab(f):
            pltpu.make_async_copy(tab_hbm.at[f, d, :], slab_v, sem_s).wait()

        def idx_src(f, h):
            return cat_hbm.at[f, pl.ds(h * BH, BH)]

        def out_dst(f, h):
            return out_hbm.at[f * D + d, pl.ds(h * BH, BH)]

        def gather_half(buf):
            def gather16(k, carry):
                iv = plsc.bitcast(buf[pl.ds(k * L, L)], jnp.int32)
                buf[pl.ds(k * L, L)] = plsc.load_gather(slab_v, [iv])
                return carry

            lax.fori_loop(0, BH // L, gather16, 0, unroll=16)

        issue_slab(0)
        pltpu.async_copy(idx_src(0, 0), buf_a, sem_ia)

        def do_field(f, carry):
            @pl.when(f > 0)
            def _():
                pltpu.make_async_copy(buf_b, out_dst(f - 1, 1), sem_ob).wait()
            pltpu.async_copy(idx_src(f, 1), buf_b, sem_ib)

            pltpu.make_async_copy(idx_src(f, 0), buf_a, sem_ia).wait()
            wait_slab(f)
            gather_half(buf_a)
            pltpu.async_copy(buf_a, out_dst(f, 0), sem_oa)

            pltpu.make_async_copy(idx_src(f, 1), buf_b, sem_ib).wait()
            gather_half(buf_b)

            @pl.when(f < F - 1)
            def _():
                issue_slab(f + 1)
            pltpu.async_copy(buf_b, out_dst(f, 1), sem_ob)

            pltpu.make_async_copy(buf_a, out_dst(f, 0), sem_oa).wait()

            @pl.when(f < F - 1)
            def _():
                pltpu.async_copy(idx_src(f + 1, 0), buf_a, sem_ia)
            return carry

        lax.fori_loop(0, F, do_field, 0)
        pltpu.make_async_copy(buf_b, out_dst(F - 1, 1), sem_ob).wait()

    return col_gather


def kernel(categorical_features, tables):
    B, F = categorical_features.shape
    Ft, V, D = tables.shape
    assert Ft == F
    cat_f32 = lax.bitcast_convert_type(
        categorical_features.astype(jnp.int32), jnp.float32)
    cat_t = cat_f32.T
    tab_t = jnp.transpose(tables, (0, 2, 1))
    out_t = _build(B, F, V, D)(cat_t, tab_t)
    return out_t.T


# speedup vs baseline: 1.5949x; 1.3344x over previous
import functools

import jax
import jax.numpy as jnp
from jax import lax
from jax.experimental import pallas as pl
from jax.experimental.pallas import tpu as pltpu
from jax.experimental.pallas import tpu_sc as plsc

NC = 2
NS = 16
NW = NC * NS
L = 16


@functools.lru_cache(maxsize=None)
def _build(B, F, V, D):
    assert D == NW, "one embedding dim per vector subcore"
    BH = B // 2
    assert BH % L == 0

    mesh = plsc.VectorSubcoreMesh(core_axis_name="c", subcore_axis_name="s")

    @functools.partial(
        pl.kernel,
        out_type=jax.ShapeDtypeStruct((F * D, B), jnp.float32),
        mesh=mesh,
        compiler_params=pltpu.CompilerParams(needs_layout_passes=False),
        scratch_types=[
            pltpu.VMEM((V,), jnp.float32),
            pltpu.VMEM((BH,), jnp.float32),
            pltpu.VMEM((BH,), jnp.float32),
            pltpu.SemaphoreType.DMA,
            pltpu.SemaphoreType.DMA,
            pltpu.SemaphoreType.DMA,
            pltpu.SemaphoreType.DMA,
            pltpu.SemaphoreType.DMA,
        ],
    )
    def col_gather(cat_hbm, tab_hbm, out_hbm, slab_v, buf_a, buf_b,
                   sem_s, sem_ia, sem_ib, sem_oa, sem_ob):
        cid = lax.axis_index("c")
        sid = lax.axis_index("s")
        d = cid * NS + sid

        def issue_slab(f):
            pltpu.async_copy(tab_hbm.at[f, d, :], slab_v, sem_s)

        def wait_slab(f):
            pltpu.make_async_copy(tab_hbm.at[f, d, :], slab_v, sem_s).wait()

        def idx_src(f, h):
            return cat_hbm.at[f, pl.ds(h * BH, BH)]

        def out_dst(f, h):
            return out_hbm.at[f * D + d, pl.ds(h * BH, BH)]

        def gather_half(buf):
            @plsc.parallel_loop(0, BH, L, unroll=16)
            def gather16(k):
                iv = plsc.bitcast(buf[pl.ds(k, L)], jnp.int32)
                buf[pl.ds(k, L)] = plsc.load_gather(slab_v, [iv])

        issue_slab(0)
        pltpu.async_copy(idx_src(0, 0), buf_a, sem_ia)

        def do_field(f, carry):
            @pl.when(f > 0)
            def _():
                pltpu.make_async_copy(buf_b, out_dst(f - 1, 1), sem_ob).wait()
            pltpu.async_copy(idx_src(f, 1), buf_b, sem_ib)

            pltpu.make_async_copy(idx_src(f, 0), buf_a, sem_ia).wait()
            wait_slab(f)
            gather_half(buf_a)
            pltpu.async_copy(buf_a, out_dst(f, 0), sem_oa)

            pltpu.make_async_copy(idx_src(f, 1), buf_b, sem_ib).wait()
            gather_half(buf_b)

            @pl.when(f < F - 1)
            def _():
                issue_slab(f + 1)
            pltpu.async_copy(buf_b, out_dst(f, 1), sem_ob)

            pltpu.make_async_copy(buf_a, out_dst(f, 0), sem_oa).wait()

            @pl.when(f < F - 1)
            def _():
                pltpu.async_copy(idx_src(f + 1, 0), buf_a, sem_ia)
            return carry

        lax.fori_loop(0, F, do_field, 0)
        pltpu.make_async_copy(buf_b, out_dst(F - 1, 1), sem_ob).wait()

    return col_gather


def kernel(categorical_features, tables):
    B, F = categorical_features.shape
    Ft, V, D = tables.shape
    assert Ft == F
    cat_f32 = lax.bitcast_convert_type(
        categorical_features.astype(jnp.int32), jnp.float32)
    cat_t = cat_f32.T
    tab_t = jnp.transpose(tables, (0, 2, 1))
    out_t = _build(B, F, V, D)(cat_t, tab_t)
    return out_t.T


# speedup vs baseline: 1.7848x; 1.1191x over previous
import functools

import jax
import jax.numpy as jnp
from jax import lax
from jax.experimental import pallas as pl
from jax.experimental.pallas import tpu as pltpu
from jax.experimental.pallas import tpu_sc as plsc

NC = 2
NS = 16
NW = NC * NS
L = 16


@functools.lru_cache(maxsize=None)
def _build(B, F, V, D):
    assert D == NW, "one embedding dim per vector subcore"
    BH = B // 2
    assert BH % L == 0
    SPLIT = (V // 2) // 128 * 128
    VHI = V - SPLIT

    mesh = plsc.VectorSubcoreMesh(core_axis_name="c", subcore_axis_name="s")

    @functools.partial(
        pl.kernel,
        out_type=jax.ShapeDtypeStruct((F * D, B), jnp.float32),
        mesh=mesh,
        compiler_params=pltpu.CompilerParams(needs_layout_passes=False),
        scratch_types=[
            pltpu.VMEM((SPLIT,), jnp.float32),
            pltpu.VMEM((VHI,), jnp.float32),
            pltpu.VMEM((BH,), jnp.float32),
            pltpu.VMEM((BH,), jnp.float32),
            pltpu.SemaphoreType.DMA,
            pltpu.SemaphoreType.DMA,
            pltpu.SemaphoreType.DMA,
            pltpu.SemaphoreType.DMA,
            pltpu.SemaphoreType.DMA,
            pltpu.SemaphoreType.DMA,
        ],
    )
    def col_gather(cat_hbm, tab_hbm, out_hbm, lo_v, hi_v, buf_a, buf_b,
                   sem_lo, sem_hi, sem_ia, sem_ib, sem_oa, sem_ob):
        cid = lax.axis_index("c")
        sid = lax.axis_index("s")
        d = cid * NS + sid

        def lo_src(f):
            return tab_hbm.at[f, d, pl.ds(0, SPLIT)]

        def hi_src(f):
            return tab_hbm.at[f, d, pl.ds(SPLIT, VHI)]

        def idx_src(f, h):
            return cat_hbm.at[f, pl.ds(h * BH, BH)]

        def out_dst(f, h):
            return out_hbm.at[f * D + d, pl.ds(h * BH, BH)]

        def pass_lo(buf):
            @plsc.parallel_loop(0, BH, L, unroll=16)
            def body(k):
                ivf = buf[pl.ds(k, L)]
                iv = plsc.bitcast(ivf, jnp.int32)
                m = iv < SPLIT
                g = plsc.load_gather(lo_v, [iv], mask=m)
                buf[pl.ds(k, L)] = lax.select(m, g, ivf)

        def pass_hi(buf):
            @plsc.parallel_loop(0, BH, L, unroll=16)
            def body(k):
                ivf = buf[pl.ds(k, L)]
                iv = plsc.bitcast(ivf, jnp.int32)
                m = (iv >= SPLIT) & (iv < V)
                g = plsc.load_gather(hi_v, [iv - SPLIT], mask=m)
                buf[pl.ds(k, L)] = lax.select(m, g, ivf)

        pltpu.async_copy(lo_src(0), lo_v, sem_lo)
        pltpu.async_copy(hi_src(0), hi_v, sem_hi)
        pltpu.async_copy(idx_src(0, 0), buf_a, sem_ia)

        def do_field(f, carry):
            @pl.when(f > 0)
            def _():
                pltpu.make_async_copy(buf_b, out_dst(f - 1, 1), sem_ob).wait()
            pltpu.async_copy(idx_src(f, 1), buf_b, sem_ib)

            pltpu.make_async_copy(idx_src(f, 0), buf_a, sem_ia).wait()
            pltpu.make_async_copy(lo_src(f), lo_v, sem_lo).wait()
            pass_lo(buf_a)
            pltpu.make_async_copy(idx_src(f, 1), buf_b, sem_ib).wait()
            pass_lo(buf_b)

            @pl.when(f < F - 1)
            def _():
                pltpu.async_copy(lo_src(f + 1), lo_v, sem_lo)

            pltpu.make_async_copy(hi_src(f), hi_v, sem_hi).wait()
            pass_hi(buf_a)
            pltpu.async_copy(buf_a, out_dst(f, 0), sem_oa)
            pass_hi(buf_b)

            @pl.when(f < F - 1)
            def _():
                pltpu.async_copy(hi_src(f + 1), hi_v, sem_hi)
            pltpu.async_copy(buf_b, out_dst(f, 1), sem_ob)

            pltpu.make_async_copy(buf_a, out_dst(f, 0), sem_oa).wait()

            @pl.when(f < F - 1)
            def _():
                pltpu.async_copy(idx_src(f + 1, 0), buf_a, sem_ia)
            return carry

        lax.fori_loop(0, F, do_field, 0)
        pltpu.make_async_copy(buf_b, out_dst(F - 1, 1), sem_ob).wait()

    return col_gather


def kernel(categorical_features, tables):
    B, F = categorical_features.shape
    Ft, V, D = tables.shape
    assert Ft == F
    cat_f32 = lax.bitcast_convert_type(
        categorical_features.astype(jnp.int32), jnp.float32)
    cat_t = cat_f32.T
    tab_t = jnp.transpose(tables, (0, 2, 1))
    out_t = _build(B, F, V, D)(cat_t, tab_t)
    return out_t.T
